# baseline (speedup 1.0000x reference)
import functools

import jax
import jax.numpy as jnp
from jax import lax
from jax.experimental import pallas as pl
from jax.experimental.pallas import tpu as pltpu
from jax.experimental.pallas import tpu_sc as plsc

N = 10000
E = 640000
D = 32
D4 = 4 * D

NC = 2
NS = 16
CHUNK = 40
EDGES_PER_CORE = E // NC
EDGES_PER_TILE = EDGES_PER_CORE // NS
N_CHUNKS = EDGES_PER_TILE // CHUNK
NPAD = 10240
ROWS_PER_TILE = NPAD // NS

NBUF = 4
LOOKAHEAD = 3

_MESH = plsc.VectorSubcoreMesh(core_axis_name="c", subcore_axis_name="s")


CH = 40
NCH = EDGES_PER_TILE // CH
NB = 5
NI = 10
LA = 4


def _sc_seg_sum(data_hbm_args, dst, zeros, gather, n_edges=E):
    edges_per_core = n_edges // NC
    edges_per_tile = edges_per_core // NS
    nch = edges_per_tile // CH
    assert nch % NI == 0 and nch >= NI
    scratch = [pltpu.VMEM_SHARED((NPAD, D4), jnp.float32)]
    scratch += [pltpu.VMEM((CH, D4), jnp.float32)] * NB
    scratch += [pltpu.VMEM((CH,), jnp.int32)] * NI
    if gather:
        scratch += [pltpu.VMEM((CH,), jnp.int32)] * NI
    scratch += [pltpu.SemaphoreType.DMA] * (2 * NB + (2 if gather else 1) * NI)

    @functools.partial(
        pl.kernel,
        out_type=jax.ShapeDtypeStruct((NC, NPAD, D4), jnp.float32),
        mesh=_MESH,
        scratch_types=scratch,
    )
    def k(*args):
        if gather:
            q_hbm, src_hbm, dst_hbm, z_hbm, out_hbm, accum = args[:6]
            bs = args[6:]
            d_v = bs[:NB]; bs = bs[NB:]
            dst_v = bs[:NI]; bs = bs[NI:]
            src_v = bs[:NI]; bs = bs[NI:]
            sem_d = bs[:NB]; sem_s = bs[NB:2 * NB]
            sem_id = bs[2 * NB:2 * NB + NI]
            sem_is = bs[2 * NB + NI:]
        else:
            ea_hbm, dst_hbm, z_hbm, out_hbm, accum = args[:5]
            bs = args[5:]
            d_v = bs[:NB]; bs = bs[NB:]
            dst_v = bs[:NI]; bs = bs[NI:]
            sem_d = bs[:NB]; sem_s = bs[NB:2 * NB]
            sem_id = bs[2 * NB:2 * NB + NI]

        c = lax.axis_index("c")
        s = lax.axis_index("s")
        pltpu.sync_copy(z_hbm, accum.at[pl.ds(s * ROWS_PER_TILE, ROWS_PER_TILE)])
        plsc.subcore_barrier()

        base0 = c * edges_per_core + s * edges_per_tile

        def start_idx(g, par):
            pltpu.async_copy(dst_hbm.at[pl.ds(base0 + g * CH, CH)],
                             dst_v[par], sem_id[par])
            if gather:
                pltpu.async_copy(src_hbm.at[pl.ds(base0 + g * CH, CH)],
                                 src_v[par], sem_is[par])

        def wait_idx(par):
            pltpu.make_async_copy(dst_hbm.at[pl.ds(0, CH)], dst_v[par],
                                  sem_id[par]).wait()
            if gather:
                pltpu.make_async_copy(dst_hbm.at[pl.ds(0, CH)], src_v[par],
                                      sem_is[par]).wait()

        def start_data(g, dpar, ipar):
            if gather:
                pltpu.async_copy(q_hbm.at[src_v[ipar]], d_v[dpar], sem_d[dpar])
            else:
                pltpu.async_copy(ea_hbm.at[pl.ds(base0 + g * CH, CH)],
                                 d_v[dpar], sem_d[dpar])

        def wait_data(dpar):
            ref = q_hbm if gather else ea_hbm
            pltpu.make_async_copy(ref.at[pl.ds(0, CH)], d_v[dpar],
                                  sem_d[dpar]).wait()

        def wait_scatter(dpar):
            ref = q_hbm if gather else ea_hbm
            pltpu.make_async_copy(ref.at[pl.ds(0, CH)], d_v[dpar],
                                  sem_s[dpar]).wait()

        for h in range(NI - 1):
            start_idx(h, h)
        for h in range(LA):
            wait_idx(h)
            start_data(h, h, h)

        def body(pp, carry):
            for j in range(NI):
                g = pp * NI + j
                ga = g + LA

                @pl.when(ga < nch)
                def _():
                    wait_idx((j + LA) % NI)

                @pl.when(g >= 1)
                def _():
                    wait_scatter((j - 1) % NB)

                @pl.when(ga < nch)
                def _():
                    start_data(ga, (j + LA) % NB, (j + LA) % NI)

                wait_data(j % NB)
                pltpu.async_copy(d_v[j % NB], accum.at[dst_v[j % NI]],
                                 sem_s[j % NB], add=True)

                @pl.when(g + NI - 1 < nch)
                def _():
                    start_idx(g + NI - 1, (j - 1) % NI)
            return carry

        lax.fori_loop(0, nch // NI, body, 0)
        wait_scatter((nch - 1) % NB)

        plsc.subcore_barrier()
        pltpu.sync_copy(accum.at[pl.ds(s * ROWS_PER_TILE, ROWS_PER_TILE)],
                        out_hbm.at[c, pl.ds(s * ROWS_PER_TILE, ROWS_PER_TILE)])

    return k(*data_hbm_args, dst, zeros)


def _tc_transform(parts, q_flat, h, b_flat):
    blk = 2000
    np_ = len(parts)

    def body(*refs):
        p_refs = refs[:np_]
        q_ref, h_ref, b_ref, o_ref = refs[np_:]
        acc = p_refs[0][0] + p_refs[0][1]
        for pr in p_refs[1:]:
            acc = acc + pr[0] + pr[1]
        o_ref[...] = (jnp.dot(acc, h_ref[...], preferred_element_type=jnp.float32)
                      + b_ref[...] + q_ref[...])

    return pl.pallas_call(
        body,
        grid=(N // blk,),
        in_specs=[pl.BlockSpec((NC, blk, D4), lambda i: (0, i, 0))] * np_ + [
            pl.BlockSpec((blk, D4), lambda i: (i, 0)),
            pl.BlockSpec((D4, D4), lambda i: (0, 0)),
            pl.BlockSpec((1, D4), lambda i: (0, 0)),
        ],
        out_specs=pl.BlockSpec((blk, D4), lambda i: (i, 0)),
        out_shape=jax.ShapeDtypeStruct((N, D4), jnp.float32),
    )(*parts, q_flat, h, b_flat)


def kernel(q, edge_index, edge_attr, W_r, W_i, W_j, W_k, b):
    q_flat = q.reshape(N, D4)
    src = edge_index[0].astype(jnp.int32)
    dst = edge_index[1].astype(jnp.int32)
    zeros = jnp.zeros((ROWS_PER_TILE, D4), jnp.float32)

    pa = _sc_seg_sum((q_flat, src), dst, zeros, gather=True)
    eh = E // 2
    parts = [pa]
    for i in range(2):
        ea_i = edge_attr[i * eh:(i + 1) * eh].reshape(eh, D4)
        parts.append(_sc_seg_sum((ea_i,), dst[i * eh:(i + 1) * eh], zeros,
                                 gather=False, n_edges=eh))

    h = jnp.concatenate([
        jnp.concatenate([W_r, W_i, W_j, W_k], axis=1),
        jnp.concatenate([-W_i, W_r, -W_k, W_j], axis=1),
        jnp.concatenate([-W_j, W_k, W_r, -W_i], axis=1),
        jnp.concatenate([-W_k, -W_j, W_i, W_r], axis=1),
    ], axis=0)
    b_flat = b.reshape(1, D4)

    out = _tc_transform(parts, q_flat, h, b_flat)
    return out.reshape(N, 4, D)


# speedup vs baseline: 1.4144x; 1.4144x over previous
import functools

import jax
import jax.numpy as jnp
from jax import lax
from jax.experimental import pallas as pl
from jax.experimental.pallas import tpu as pltpu
from jax.experimental.pallas import tpu_sc as plsc

N = 10000
E = 640000
D = 32
D4 = 4 * D

NC = 2
NS = 16
CHUNK = 40
EDGES_PER_CORE = E // NC
EDGES_PER_TILE = EDGES_PER_CORE // NS
N_CHUNKS = EDGES_PER_TILE // CHUNK
NPAD = 10240
ROWS_PER_TILE = NPAD // NS

NBUF = 4
LOOKAHEAD = 3

_MESH = plsc.VectorSubcoreMesh(core_axis_name="c", subcore_axis_name="s")


CH = 40
NCH = EDGES_PER_TILE // CH
NB = 5
NI = 10
LA = 4


def _sc_seg_sum(data_hbm_args, dst, zeros, gather, n_edges=E):
    edges_per_core = n_edges // NC
    edges_per_tile = edges_per_core // NS
    nch = edges_per_tile // CH
    assert nch % NI == 0 and nch >= NI
    scratch = [pltpu.VMEM_SHARED((NPAD, D4), jnp.float32)]
    scratch += [pltpu.VMEM((CH, D4), jnp.float32)] * NB
    scratch += [pltpu.VMEM((CH,), jnp.int32)] * NI
    if gather:
        scratch += [pltpu.VMEM((CH,), jnp.int32)] * NI
    scratch += [pltpu.SemaphoreType.DMA] * (2 * NB + (2 if gather else 1) * NI)

    @functools.partial(
        pl.kernel,
        out_type=jax.ShapeDtypeStruct((NC, NPAD, D4), jnp.float32),
        mesh=_MESH,
        scratch_types=scratch,
    )
    def k(*args):
        if gather:
            q_hbm, src_hbm, dst_hbm, z_hbm, out_hbm, accum = args[:6]
            bs = args[6:]
            d_v = bs[:NB]; bs = bs[NB:]
            dst_v = bs[:NI]; bs = bs[NI:]
            src_v = bs[:NI]; bs = bs[NI:]
            sem_d = bs[:NB]; sem_s = bs[NB:2 * NB]
            sem_id = bs[2 * NB:2 * NB + NI]
            sem_is = bs[2 * NB + NI:]
        else:
            ea_hbm, dst_hbm, z_hbm, out_hbm, accum = args[:5]
            bs = args[5:]
            d_v = bs[:NB]; bs = bs[NB:]
            dst_v = bs[:NI]; bs = bs[NI:]
            sem_d = bs[:NB]; sem_s = bs[NB:2 * NB]
            sem_id = bs[2 * NB:2 * NB + NI]

        c = lax.axis_index("c")
        s = lax.axis_index("s")
        pltpu.sync_copy(z_hbm, accum.at[pl.ds(s * ROWS_PER_TILE, ROWS_PER_TILE)])
        plsc.subcore_barrier()

        base0 = c * edges_per_core + s * edges_per_tile

        def start_idx(g, par):
            pltpu.async_copy(dst_hbm.at[pl.ds(base0 + g * CH, CH)],
                             dst_v[par], sem_id[par])
            if gather:
                pltpu.async_copy(src_hbm.at[pl.ds(base0 + g * CH, CH)],
                                 src_v[par], sem_is[par])

        def wait_idx(par):
            pltpu.make_async_copy(dst_hbm.at[pl.ds(0, CH)], dst_v[par],
                                  sem_id[par]).wait()
            if gather:
                pltpu.make_async_copy(dst_hbm.at[pl.ds(0, CH)], src_v[par],
                                      sem_is[par]).wait()

        def start_data(g, dpar, ipar):
            if gather:
                pltpu.async_copy(q_hbm.at[src_v[ipar]], d_v[dpar], sem_d[dpar])
            else:
                pltpu.async_copy(ea_hbm.at[pl.ds(base0 + g * CH, CH)],
                                 d_v[dpar], sem_d[dpar])

        def wait_data(dpar):
            ref = q_hbm if gather else ea_hbm
            pltpu.make_async_copy(ref.at[pl.ds(0, CH)], d_v[dpar],
                                  sem_d[dpar]).wait()

        def wait_scatter(dpar):
            ref = q_hbm if gather else ea_hbm
            pltpu.make_async_copy(ref.at[pl.ds(0, CH)], d_v[dpar],
                                  sem_s[dpar]).wait()

        for h in range(NI - 1):
            start_idx(h, h)
        for h in range(LA):
            wait_idx(h)
            start_data(h, h, h)

        def body(pp, carry):
            for j in range(NI):
                g = pp * NI + j
                ga = g + LA

                @pl.when(ga < nch)
                def _():
                    wait_idx((j + LA) % NI)

                @pl.when(g >= 1)
                def _():
                    wait_scatter((j - 1) % NB)

                @pl.when(ga < nch)
                def _():
                    start_data(ga, (j + LA) % NB, (j + LA) % NI)

                wait_data(j % NB)
                pltpu.async_copy(d_v[j % NB], accum.at[dst_v[j % NI]],
                                 sem_s[j % NB], add=True)

                @pl.when(g + NI - 1 < nch)
                def _():
                    start_idx(g + NI - 1, (j - 1) % NI)
            return carry

        lax.fori_loop(0, nch // NI, body, 0)
        wait_scatter((nch - 1) % NB)

        plsc.subcore_barrier()
        pltpu.sync_copy(accum.at[pl.ds(s * ROWS_PER_TILE, ROWS_PER_TILE)],
                        out_hbm.at[c, pl.ds(s * ROWS_PER_TILE, ROWS_PER_TILE)])

    return k(*data_hbm_args, dst, zeros)


def _tc_transform(parts, q_flat, h, b_flat):
    blk = 2000
    np_ = len(parts)

    def body(*refs):
        p_refs = refs[:np_]
        q_ref, h_ref, b_ref, o_ref = refs[np_:]
        acc = p_refs[0][0] + p_refs[0][1]
        for pr in p_refs[1:]:
            acc = acc + pr[0] + pr[1]
        o_ref[...] = (jnp.dot(acc, h_ref[...], preferred_element_type=jnp.float32)
                      + b_ref[...] + q_ref[...])

    return pl.pallas_call(
        body,
        grid=(N // blk,),
        in_specs=[pl.BlockSpec((NC, blk, D4), lambda i: (0, i, 0))] * np_ + [
            pl.BlockSpec((blk, D4), lambda i: (i, 0)),
            pl.BlockSpec((D4, D4), lambda i: (0, 0)),
            pl.BlockSpec((1, D4), lambda i: (0, 0)),
        ],
        out_specs=pl.BlockSpec((blk, D4), lambda i: (i, 0)),
        out_shape=jax.ShapeDtypeStruct((N, D4), jnp.float32),
    )(*parts, q_flat, h, b_flat)


def kernel(q, edge_index, edge_attr, W_r, W_i, W_j, W_k, b):
    q_flat = q.reshape(N, D4)
    src = edge_index[0].astype(jnp.int32)
    dst = edge_index[1].astype(jnp.int32)
    zeros = jnp.zeros((ROWS_PER_TILE, D4), jnp.float32)

    pa = _sc_seg_sum((q_flat, src), dst, zeros, gather=True)
    ea_flat = edge_attr.reshape(E, D4)
    pb = _sc_seg_sum((ea_flat,), dst, zeros, gather=False)
    parts = [pa, pb]

    h = jnp.concatenate([
        jnp.concatenate([W_r, W_i, W_j, W_k], axis=1),
        jnp.concatenate([-W_i, W_r, -W_k, W_j], axis=1),
        jnp.concatenate([-W_j, W_k, W_r, -W_i], axis=1),
        jnp.concatenate([-W_k, -W_j, W_i, W_r], axis=1),
    ], axis=0)
    b_flat = b.reshape(1, D4)

    out = _tc_transform(parts, q_flat, h, b_flat)
    return out.reshape(N, 4, D)
